# Initial kernel scaffold; baseline (speedup 1.0000x reference)
import functools

import jax
import jax.numpy as jnp
import numpy as np
from jax.experimental import pallas as pl

N = 10000
E = 160000
NUM_SPECIES = 10
F = 128
NB = 8
SH = 9
R_MAX = 5.0
EPS = 0.5
HR = 64
HRO = 16

NODE_BLK = 400


def _node_phase_body(a_ref, cw_ref, wlin_ref, out_ref):
    A = a_ref[...]
    scal = jnp.zeros((NODE_BLK, F), jnp.float32)
    for a in range(SH):
        blk = A[:, a * F:(a + 1) * F]
        scal = scal + blk * blk
    cw = cw_ref[...]
    g = cw[:, 0:F] + cw[:, F:2 * F] * scal + cw[:, 2 * F:3 * F] * (scal * scal)
    b0 = A[:, 0:F] * g
    out_ref[...] = jnp.dot(b0, wlin_ref[...], preferred_element_type=jnp.float32)


def _node_phase(A, cw, Wlin):
    grid = (N // NODE_BLK,)
    return pl.pallas_call(
        _node_phase_body,
        grid=grid,
        in_specs=[
            pl.BlockSpec((NODE_BLK, SH * F), lambda i: (i, 0)),
            pl.BlockSpec((NODE_BLK, 3 * F), lambda i: (i, 0)),
            pl.BlockSpec((F, F), lambda i: (0, 0)),
        ],
        out_specs=pl.BlockSpec((NODE_BLK, F), lambda i: (i, 0)),
        out_shape=jax.ShapeDtypeStruct((N, F), jnp.float32),
    )(A, cw, Wlin)


def _sph(u):
    x, y, z = u[:, 0], u[:, 1], u[:, 2]
    s3 = float(np.sqrt(3.0)); s15 = float(np.sqrt(15.0)); s5 = float(np.sqrt(5.0))
    comps = [jnp.ones_like(x), s3 * x, s3 * y, s3 * z,
             s15 * x * y, s15 * y * z, 0.5 * s5 * (3.0 * z * z - 1.0),
             s15 * x * z, 0.5 * s15 * (x * x - y * y)]
    return jnp.stack(comps, axis=-1)


def _radial(r):
    n = jnp.arange(1, NB + 1, dtype=jnp.float32)
    rs = jnp.clip(r, 1e-9, None)
    rb = np.sqrt(2.0 / R_MAX) * jnp.sin(n * jnp.pi * rs / R_MAX) / rs
    x = r / R_MAX
    env = 1.0 - 21.0 * x ** 5 + 35.0 * x ** 6 - 15.0 * x ** 7
    env = jnp.where(x < 1.0, env, 0.0)
    return rb * env


def kernel(vectors, node_specie, senders, receivers, W_embed, W_up0, Wr1_0, Wr2_0, Wc0, Wlin0, Wro0, W_up1, Wr1_1, Wr2_1, Wc1, Wsc_lin1, Wsc_sp1, Wlin1, Wro1a, Wro1b):
    lengths = jnp.sqrt(jnp.sum(vectors * vectors, axis=-1, keepdims=True) + 1e-12)
    Y = _sph(vectors / lengths)
    ef = _radial(lengths)
    yr0 = Y * (jax.nn.silu(ef @ Wr1_0) @ Wr2_0)
    yr1 = Y * (jax.nn.silu(ef @ Wr1_1) @ Wr2_1)

    onehot = jax.nn.one_hot(node_specie, NUM_SPECIES, dtype=jnp.float32)
    h0 = onehot @ (W_embed @ W_up0)
    cw0 = onehot @ Wc0.reshape(NUM_SPECIES, 3 * F)
    cw1 = onehot @ Wc1.reshape(NUM_SPECIES, 3 * F)

    def edge_phase(h, yr):
        s = h[senders]
        m = (yr[:, :, None] * s[:, None, :]).reshape(E, SH * F)
        A = jnp.zeros((N, SH * F), jnp.float32).at[receivers].add(m)
        return A * EPS

    A1 = edge_phase(h0, yr0)
    nf1_0 = _node_phase(A1, cw0, Wlin0)
    ro0 = nf1_0 @ Wro0

    h1 = nf1_0 @ W_up1
    A2 = edge_phase(h1, yr1)
    nf2_0 = _node_phase(A2, cw1, Wlin1)
    nf2_0 = nf2_0 + (nf1_0 @ Wsc_lin1) * (onehot @ Wsc_sp1)
    ro1 = jax.nn.silu(nf2_0 @ Wro1a) @ Wro1b
    return jnp.stack([ro0, ro1], axis=1)


# speedup vs baseline: 7.4394x; 7.4394x over previous
import functools

import jax
import jax.numpy as jnp
import numpy as np
from jax.experimental import pallas as pl

N = 10000
E = 160000
NUM_SPECIES = 10
F = 128
NB = 8
SH = 9
R_MAX = 5.0
EPS = 0.5
HR = 64
HRO = 16

NODE_BLK = 400


def _node_phase_body(a_ref, cw_ref, wlin_ref, out_ref):
    A = a_ref[...]
    scal = jnp.zeros((NODE_BLK, F), jnp.float32)
    for a in range(SH):
        blk = A[:, a * F:(a + 1) * F]
        scal = scal + blk * blk
    cw = cw_ref[...]
    g = cw[:, 0:F] + cw[:, F:2 * F] * scal + cw[:, 2 * F:3 * F] * (scal * scal)
    b0 = A[:, 0:F] * g
    out_ref[...] = jnp.dot(b0, wlin_ref[...], preferred_element_type=jnp.float32)


def _node_phase(A, cw, Wlin):
    grid = (N // NODE_BLK,)
    return pl.pallas_call(
        _node_phase_body,
        grid=grid,
        in_specs=[
            pl.BlockSpec((NODE_BLK, SH * F), lambda i: (i, 0)),
            pl.BlockSpec((NODE_BLK, 3 * F), lambda i: (i, 0)),
            pl.BlockSpec((F, F), lambda i: (0, 0)),
        ],
        out_specs=pl.BlockSpec((NODE_BLK, F), lambda i: (i, 0)),
        out_shape=jax.ShapeDtypeStruct((N, F), jnp.float32),
    )(A, cw, Wlin)


def _sph(u):
    x, y, z = u[:, 0], u[:, 1], u[:, 2]
    s3 = float(np.sqrt(3.0)); s15 = float(np.sqrt(15.0)); s5 = float(np.sqrt(5.0))
    comps = [jnp.ones_like(x), s3 * x, s3 * y, s3 * z,
             s15 * x * y, s15 * y * z, 0.5 * s5 * (3.0 * z * z - 1.0),
             s15 * x * z, 0.5 * s15 * (x * x - y * y)]
    return jnp.stack(comps, axis=-1)


def _radial(r):
    n = jnp.arange(1, NB + 1, dtype=jnp.float32)
    rs = jnp.clip(r, 1e-9, None)
    rb = np.sqrt(2.0 / R_MAX) * jnp.sin(n * jnp.pi * rs / R_MAX) / rs
    x = r / R_MAX
    env = 1.0 - 21.0 * x ** 5 + 35.0 * x ** 6 - 15.0 * x ** 7
    env = jnp.where(x < 1.0, env, 0.0)
    return rb * env


def kernel(vectors, node_specie, senders, receivers, W_embed, W_up0, Wr1_0, Wr2_0, Wc0, Wlin0, Wro0, W_up1, Wr1_1, Wr2_1, Wc1, Wsc_lin1, Wsc_sp1, Wlin1, Wro1a, Wro1b):
    lengths = jnp.sqrt(jnp.sum(vectors * vectors, axis=-1, keepdims=True) + 1e-12)
    Y = _sph(vectors / lengths)
    ef = _radial(lengths)
    yr0 = Y * (jax.nn.silu(ef @ Wr1_0) @ Wr2_0)
    yr1 = Y * (jax.nn.silu(ef @ Wr1_1) @ Wr2_1)

    emb = W_embed[node_specie]
    h0 = emb @ W_up0
    cw0 = Wc0[node_specie].reshape(N, 3 * F)
    cw1 = Wc1[node_specie].reshape(N, 3 * F)

    def edge_phase(h, yr):
        s = h[senders]
        m = (yr[:, :, None] * s[:, None, :]).reshape(E, SH * F)
        A = jnp.zeros((N, SH * F), jnp.float32).at[receivers].add(m)
        return A * EPS

    A1 = edge_phase(h0, yr0)
    nf1_0 = _node_phase(A1, cw0, Wlin0)
    ro0 = nf1_0 @ Wro0

    h1 = nf1_0 @ W_up1
    A2 = edge_phase(h1, yr1)
    nf2_0 = _node_phase(A2, cw1, Wlin1)
    nf2_0 = nf2_0 + (nf1_0 @ Wsc_lin1) * Wsc_sp1[node_specie]
    ro1 = jax.nn.silu(nf2_0 @ Wro1a) @ Wro1b
    return jnp.stack([ro0, ro1], axis=1)
